# baseline (speedup 1.0000x reference)
import functools

import jax
import jax.numpy as jnp
from jax.experimental import pallas as pl
from jax.experimental.pallas import tpu as pltpu

NUM_VARS = 128
K = 5
TT = 2048


def _conv_body(xc_ref, xh_ref, w_ref, b_ref, out_hbm, xs_ref, sem):
    i = pl.program_id(0)
    j = pl.program_id(1)
    nt = pl.num_programs(1)
    lin = i * nt + j
    total = pl.num_programs(0) * nt
    slot = jax.lax.rem(lin, 2)

    def _wait_for(lin2):
        i2 = jax.lax.div(lin2, nt)
        j2 = jax.lax.rem(lin2, nt)
        slot2 = jax.lax.rem(lin2, 2)

        @pl.when(j2 == 0)
        def _():
            pltpu.make_async_copy(
                xs_ref.at[slot2, 1:TT],
                out_hbm.at[i2, :TT - 1, 0, :],
                sem.at[slot2]).wait()

        @pl.when(j2 != 0)
        def _():
            pltpu.make_async_copy(
                xs_ref.at[slot2, 0:TT],
                out_hbm.at[i2, pl.ds(j2 * TT - 1, TT), 0, :],
                sem.at[slot2]).wait()

    @pl.when(lin >= 2)
    def _():
        _wait_for(lin - 2)

    halo = jnp.where(j == 0, 0.0, xh_ref[0, 4:8, :]).astype(jnp.bfloat16)
    xp = jnp.concatenate([halo, xc_ref[0].astype(jnp.bfloat16)], axis=0)
    acc = jnp.broadcast_to(b_ref[0][None, :], (TT, NUM_VARS)).astype(jnp.float32)
    for k in range(K):
        wk = w_ref[k].astype(jnp.bfloat16)
        if k == K - 1:
            row = jax.lax.broadcasted_iota(jnp.int32, (NUM_VARS, NUM_VARS), 0)
            col = jax.lax.broadcasted_iota(jnp.int32, (NUM_VARS, NUM_VARS), 1)
            wk = jnp.where(row == col, 0.0, wk)
        acc = acc + jax.lax.dot_general(
            xp[k:k + TT], wk,
            dimension_numbers=(((1,), (1,)), ((), ())),
            preferred_element_type=jnp.float32)
    xs_ref[slot] = acc

    @pl.when(j == 0)
    def _():
        pltpu.make_async_copy(
            xs_ref.at[slot, 1:TT],
            out_hbm.at[i, :TT - 1, 0, :],
            sem.at[slot]).start()

    @pl.when(j != 0)
    def _():
        pltpu.make_async_copy(
            xs_ref.at[slot, 0:TT],
            out_hbm.at[i, pl.ds(j * TT - 1, TT), 0, :],
            sem.at[slot]).start()

    @pl.when(lin == total - 1)
    def _():
        _wait_for(lin - 1)
        _wait_for(lin)


@functools.partial(jax.jit, static_argnames=())
def kernel(spikes, weight, bias):
    b, t, n, _ = spikes.shape
    nt = t // TT
    ttb = TT // 8
    x = jnp.reshape(spikes, (b, t, n))
    w = jnp.transpose(weight, (2, 0, 1))
    bias2 = bias[None, :]
    out = pl.pallas_call(
        _conv_body,
        grid=(b, nt),
        in_specs=[
            pl.BlockSpec((1, TT, n), lambda i, j: (i, j, 0)),
            pl.BlockSpec((1, 8, n),
                         lambda i, j: (i, jnp.maximum(j * ttb - 1, 0), 0)),
            pl.BlockSpec((K, n, n), lambda i, j: (0, 0, 0)),
            pl.BlockSpec((1, n), lambda i, j: (0, 0)),
        ],
        out_specs=pl.BlockSpec(memory_space=pl.ANY),
        out_shape=jax.ShapeDtypeStruct((b, t - 1, 1, n), jnp.float32),
        scratch_shapes=[
            pltpu.MemorySpace.VMEM((2, TT, n), jnp.float32),
            pltpu.SemaphoreType.DMA((2,)),
        ],
    )(x, x, w, bias2)
    return jnp.reshape(out, (b, t - 1, n, 1))
